# baseline (speedup 1.0000x reference)
import functools

import jax
import jax.numpy as jnp
import numpy as np
from jax import lax
from jax.experimental import pallas as pl
from jax.experimental.pallas import tpu as pltpu
from jax.experimental.pallas import tpu_sc as plsc

_MAP_W = 500
_NPIX = 256 * 512
_QCAP = 32768


def _resize_feat(x):
    T, C, H, W = x.shape
    ys = jnp.linspace(0.0, H - 1.0, 1024)[::4]
    xs = jnp.linspace(0.0, W - 1.0, 2048)[::4]
    y0 = jnp.floor(ys).astype(jnp.int32)
    y1 = jnp.minimum(y0 + 1, H - 1)
    x0 = jnp.floor(xs).astype(jnp.int32)
    x1 = jnp.minimum(x0 + 1, W - 1)
    wy = (ys - y0.astype(x.dtype))[None, None, :, None]
    wx = (xs - x0.astype(x.dtype))[None, None, None, :]
    top = jnp.take(x, y0, axis=2)
    bot = jnp.take(x, y1, axis=2)
    v = top * (1.0 - wy) + bot * wy
    left = jnp.take(v, x0, axis=3)
    right = jnp.take(v, x1, axis=3)
    return left * (1.0 - wx) + right * wx


def _proj_body(rows_ref, mask_ref, wt_ref, b_ref, out_ref):
    x = rows_ref[0, 0]
    msk = mask_ref[0, 0]
    tmp = jnp.dot(x, wt_ref[...], preferred_element_type=jnp.float32)
    tmp = tmp + b_ref[...]
    out_ref[0, 0] = tmp * msk


def _project_mask(rows, mask_f, W_lin, b_lin):
    T, N, C = rows.shape
    BLK = 2500
    nb = N // BLK
    rows4 = rows.reshape(T, nb, BLK, C)
    mask4 = mask_f.reshape(T, nb, BLK, 1)
    out = pl.pallas_call(
        _proj_body,
        grid=(T, nb),
        in_specs=[
            pl.BlockSpec((1, 1, BLK, C), lambda t, i: (t, i, 0, 0)),
            pl.BlockSpec((1, 1, BLK, 1), lambda t, i: (t, i, 0, 0)),
            pl.BlockSpec((C, 64), lambda t, i: (0, 0)),
            pl.BlockSpec((64,), lambda t, i: (0,)),
        ],
        out_specs=pl.BlockSpec((1, 1, BLK, 64), lambda t, i: (t, i, 0, 0)),
        out_shape=jax.ShapeDtypeStruct((T, nb, BLK, 64), jnp.float32),
    )(rows4, mask4, W_lin.T, b_lin)
    return out.reshape(T, N, 64)


_NW = 32
_CW = 7816
_NCELL = _MAP_W * _MAP_W


def _sc_gather(q_pad, pos, table_flat):
    T = q_pad.shape[0]
    qn = q_pad.shape[1]
    q_flat = q_pad.reshape(-1)
    pos_flat = pos.reshape(-1)
    mesh = plsc.VectorSubcoreMesh(core_axis_name="c", subcore_axis_name="s")

    @functools.partial(
        pl.kernel, mesh=mesh,
        out_type=jax.ShapeDtypeStruct((T * _NCELL, 32), jnp.float32),
        compiler_params=pltpu.CompilerParams(
            needs_layout_passes=False, use_tc_tiling_on_sc=False),
        scratch_types=[
            pltpu.VMEM((32768,), jnp.int32),
            pltpu.VMEM((7824,), jnp.int32),
            pltpu.VMEM((61, 128), jnp.int32),
            pltpu.VMEM((32,), jnp.int32),
            pltpu.VMEM((128, 32), jnp.float32),
            pltpu.VMEM((24, 32), jnp.float32),
            pltpu.SemaphoreType.DMA,
        ],
    )
    def k(q_hbm, pos_hbm, tab_hbm, out_hbm, pos_v, q_v, g_v, g_t, buf, tbuf, sem):
        cid = lax.axis_index("c")
        sid = lax.axis_index("s")
        wid = sid * 2 + cid
        base = wid * _CW
        for t in range(T):
            toff = t * 131072
            obase = t * _NCELL + base
            pltpu.sync_copy(pos_hbm.at[pl.ds(t * _QCAP, _QCAP)], pos_v)
            pltpu.sync_copy(q_hbm.at[pl.ds(t * qn + base, _CW)],
                            q_v.at[pl.ds(0, _CW)])

            def chunk(c, carry):
                for kk in range(8):
                    qv = q_v[pl.ds(c * 128 + kk * 16, 16)]
                    gv = plsc.load_gather(pos_v, [qv]) + toff
                    g_v[c, pl.ds(kk * 16, 16)] = gv
                pltpu.async_copy(tab_hbm.at[g_v.at[c]], buf, sem).wait()
                pltpu.sync_copy(buf, out_hbm.at[pl.ds(obase + c * 128, 128)])
                return carry

            lax.fori_loop(0, 60, chunk, 0, unroll=False)

            @pl.when(wid < _NW - 1)
            def _():
                chunk(60, 0)
                qv = jnp.clip(q_v[pl.ds(7808, 16)], 0, _QCAP - 1)
                gv = plsc.load_gather(pos_v, [qv]) + toff
                g_t[pl.ds(0, 16)] = gv
                pltpu.async_copy(tab_hbm.at[g_t.at[pl.ds(0, 8)]],
                                 tbuf.at[pl.ds(0, 8)], sem).wait()
                pltpu.sync_copy(tbuf.at[pl.ds(0, 8)],
                                out_hbm.at[pl.ds(obase + 7808, 8)])

            @pl.when(wid == _NW - 1)
            def _():
                for kk in range(2):
                    qv = jnp.clip(q_v[pl.ds(7680 + kk * 16, 16)], 0, _QCAP - 1)
                    gv = plsc.load_gather(pos_v, [qv]) + toff
                    g_t[pl.ds(kk * 16, 16)] = gv
                pltpu.async_copy(tab_hbm.at[g_t.at[pl.ds(0, 24)]], tbuf, sem).wait()
                pltpu.sync_copy(tbuf, out_hbm.at[pl.ds(obase + 7680, 24)])

    return k(q_flat, pos_flat, table_flat).reshape(T, _NCELL, 32)


def _conv2d(x, w, pad):
    return jax.lax.conv_general_dilated(
        x, jnp.transpose(w, (2, 3, 1, 0)), (1, 1), [(pad, pad), (pad, pad)],
        dimension_numbers=('NHWC', 'HWIO', 'NHWC'))


def _bn(x, g, b):
    mu = jnp.mean(x, axis=(0, 1, 2), keepdims=True)
    var = jnp.var(x, axis=(0, 1, 2), keepdims=True)
    return (x - mu) * jax.lax.rsqrt(var + 1e-5) * g[None, None, None, :] + b[None, None, None, :]


def _conv_bn_relu(x, w, g, b, pad):
    ys = [_conv2d(x[t:t + 1], w, pad) for t in range(x.shape[0])]
    y = jnp.concatenate(ys, axis=0)
    return jax.nn.relu(_bn(y, g, b))


def _decoder(mem, w1, g1, b1, w2, g2, b2, w3, g3, b3, w4, g4, b4, w5, b5):
    h = _conv_bn_relu(mem, w1, g1, b1, 3)
    h = _conv_bn_relu(h, w2, g2, b2, 1)
    h = _conv_bn_relu(h, w3, g3, b3, 1)
    h = _conv_bn_relu(h, w4, g4, b4, 1)
    ys = [_conv2d(h[t:t + 1], w5, 0) for t in range(h.shape[0])]
    return jnp.concatenate(ys, axis=0) + b5[None, None, None, :]


def kernel(features, proj_indices, masks_inliers, W_lin, b_lin, w1, g1, bb1,
           w2, g2, bb2, w3, g3, bb3, w4, g4, bb4, w5, b5):
    T = features.shape[1]
    thr = jnp.max(proj_indices, axis=1, keepdims=True)
    m = proj_indices < thr

    feat = _resize_feat(features[0])
    feat = jnp.transpose(feat, (0, 2, 3, 1))
    feat_flat = feat.reshape(T, _NPIX, 32)

    mflat = masks_inliers.reshape(T, -1).astype(jnp.int32)
    rank = jnp.cumsum(mflat, axis=1) - mflat
    n_inl = jnp.sum(mflat, axis=1)
    j = jnp.arange(_NPIX, dtype=jnp.int32)
    scat_idx = jnp.where((mflat > 0) & (rank < _QCAP), rank, _QCAP)
    pos = jnp.zeros((T, _QCAP + 1), jnp.int32)
    pos = pos.at[jnp.arange(T)[:, None], scat_idx].set(
        jnp.broadcast_to(j[None, :], (T, _NPIX)), mode='drop')

    q = jnp.clip(jnp.minimum(proj_indices, (n_inl - 1)[:, None]), 0, _QCAP - 1)
    q_pad = jnp.pad(q.astype(jnp.int32), ((0, 0), (0, _NW * _CW - _NCELL)))
    rows = _sc_gather(q_pad, pos[:, :_QCAP], feat_flat.reshape(T * _NPIX, 32))

    state = _project_mask(rows, m.astype(jnp.float32), W_lin, b_lin)
    memory = state.reshape(T, _MAP_W, _MAP_W, 64)
    semmap = _decoder(memory, w1, g1, bb1, w2, g2, bb2, w3, g3, bb3,
                      w4, g4, bb4, w5, b5)
    semmap = jnp.transpose(semmap, (0, 3, 1, 2))
    observed_masks = m.reshape(T, _MAP_W, _MAP_W)
    return (semmap, observed_masks)


# speedup vs baseline: 1.9277x; 1.9277x over previous
import functools

import jax
import jax.numpy as jnp
import numpy as np
from jax import lax
from jax.experimental import pallas as pl
from jax.experimental.pallas import tpu as pltpu
from jax.experimental.pallas import tpu_sc as plsc

_MAP_W = 500
_NPIX = 256 * 512
_QCAP = 32768


def _resize_feat(x):
    T, C, H, W = x.shape
    ys = jnp.linspace(0.0, H - 1.0, 1024)[::4]
    xs = jnp.linspace(0.0, W - 1.0, 2048)[::4]
    y0 = jnp.floor(ys).astype(jnp.int32)
    y1 = jnp.minimum(y0 + 1, H - 1)
    x0 = jnp.floor(xs).astype(jnp.int32)
    x1 = jnp.minimum(x0 + 1, W - 1)
    wy = (ys - y0.astype(x.dtype))[None, None, :, None]
    wx = (xs - x0.astype(x.dtype))[None, None, None, :]
    top = jnp.take(x, y0, axis=2)
    bot = jnp.take(x, y1, axis=2)
    v = top * (1.0 - wy) + bot * wy
    left = jnp.take(v, x0, axis=3)
    right = jnp.take(v, x1, axis=3)
    return left * (1.0 - wx) + right * wx


def _proj_body(rows_ref, mask_ref, wt_ref, b_ref, out_ref):
    x = rows_ref[0, 0]
    msk = mask_ref[0, 0]
    tmp = jnp.dot(x, wt_ref[...], preferred_element_type=jnp.float32)
    tmp = (tmp + b_ref[...]) * msk
    t3 = tmp.reshape(5, 500, 64)
    z = jnp.zeros((5, 4, 64), jnp.float32)
    z2 = jnp.zeros((5, 8, 64), jnp.float32)
    out_ref[0] = jnp.concatenate([z, t3, z2], axis=1).astype(jnp.bfloat16)


def _project_mask(rows, mask_f, W_lin, b_lin):
    T, N, C = rows.shape
    BLK = 2500
    nb = N // BLK
    rows4 = rows.reshape(T, nb, BLK, C)
    mask4 = mask_f.reshape(T, nb, BLK, 1)
    out = pl.pallas_call(
        _proj_body,
        grid=(T, nb),
        in_specs=[
            pl.BlockSpec((1, 1, BLK, C), lambda t, i: (t, i, 0, 0)),
            pl.BlockSpec((1, 1, BLK, 1), lambda t, i: (t, i, 0, 0)),
            pl.BlockSpec((C, 64), lambda t, i: (0, 0)),
            pl.BlockSpec((64,), lambda t, i: (0,)),
        ],
        out_specs=pl.BlockSpec((1, 5, 512, 64), lambda t, i: (t, i, 0, 0)),
        out_shape=jax.ShapeDtypeStruct((T, _MAP_W, 512, 64), jnp.bfloat16),
    )(rows4, mask4, W_lin.T, b_lin)
    return out


_NW = 32
_CW = 7816
_NCELL = _MAP_W * _MAP_W


def _sc_gather(q_pad, pos, table_flat):
    T = q_pad.shape[0]
    qn = q_pad.shape[1]
    q_flat = q_pad.reshape(-1)
    pos_flat = pos.reshape(-1)
    mesh = plsc.VectorSubcoreMesh(core_axis_name="c", subcore_axis_name="s")

    @functools.partial(
        pl.kernel, mesh=mesh,
        out_type=jax.ShapeDtypeStruct((T * _NCELL, 32), jnp.float32),
        compiler_params=pltpu.CompilerParams(
            needs_layout_passes=False, use_tc_tiling_on_sc=False),
        scratch_types=[
            pltpu.VMEM((32768,), jnp.int32),
            pltpu.VMEM((7824,), jnp.int32),
            pltpu.VMEM((61, 128), jnp.int32),
            pltpu.VMEM((32,), jnp.int32),
            pltpu.VMEM((128, 32), jnp.float32),
            pltpu.VMEM((24, 32), jnp.float32),
            pltpu.SemaphoreType.DMA,
        ],
    )
    def k(q_hbm, pos_hbm, tab_hbm, out_hbm, pos_v, q_v, g_v, g_t, buf, tbuf, sem):
        cid = lax.axis_index("c")
        sid = lax.axis_index("s")
        wid = sid * 2 + cid
        base = wid * _CW
        for t in range(T):
            toff = t * 131072
            obase = t * _NCELL + base
            pltpu.sync_copy(pos_hbm.at[pl.ds(t * _QCAP, _QCAP)], pos_v)
            pltpu.sync_copy(q_hbm.at[pl.ds(t * qn + base, _CW)],
                            q_v.at[pl.ds(0, _CW)])

            def chunk(c, carry):
                for kk in range(8):
                    qv = q_v[pl.ds(c * 128 + kk * 16, 16)]
                    gv = plsc.load_gather(pos_v, [qv]) + toff
                    g_v[c, pl.ds(kk * 16, 16)] = gv
                pltpu.async_copy(tab_hbm.at[g_v.at[c]], buf, sem).wait()
                pltpu.sync_copy(buf, out_hbm.at[pl.ds(obase + c * 128, 128)])
                return carry

            lax.fori_loop(0, 60, chunk, 0, unroll=False)

            @pl.when(wid < _NW - 1)
            def _():
                chunk(60, 0)
                qv = jnp.clip(q_v[pl.ds(7808, 16)], 0, _QCAP - 1)
                gv = plsc.load_gather(pos_v, [qv]) + toff
                g_t[pl.ds(0, 16)] = gv
                pltpu.async_copy(tab_hbm.at[g_t.at[pl.ds(0, 8)]],
                                 tbuf.at[pl.ds(0, 8)], sem).wait()
                pltpu.sync_copy(tbuf.at[pl.ds(0, 8)],
                                out_hbm.at[pl.ds(obase + 7808, 8)])

            @pl.when(wid == _NW - 1)
            def _():
                for kk in range(2):
                    qv = jnp.clip(q_v[pl.ds(7680 + kk * 16, 16)], 0, _QCAP - 1)
                    gv = plsc.load_gather(pos_v, [qv]) + toff
                    g_t[pl.ds(kk * 16, 16)] = gv
                pltpu.async_copy(tab_hbm.at[g_t.at[pl.ds(0, 24)]], tbuf, sem).wait()
                pltpu.sync_copy(tbuf, out_hbm.at[pl.ds(obase + 7680, 24)])

    return k(q_flat, pos_flat, table_flat).reshape(T, _NCELL, 32)


_R = 10
_NB = _MAP_W // _R


def _prep_w(w, ksize, cin_p, cout_p):
    O, I = w.shape[0], w.shape[1]
    wt = jnp.transpose(w, (2, 3, 1, 0))
    wt = jnp.pad(wt, ((0, 0), (0, 0), (0, cin_p - I), (0, cout_p - O)))
    k2 = wt.reshape(ksize, ksize * cin_p, cout_p)
    Kp = ((ksize * cin_p + 127) // 128) * 128
    k2 = jnp.pad(k2, ((0, 0), (0, Kp - ksize * cin_p), (0, 0)))
    return k2.astype(jnp.bfloat16)


def _conv_layer(y_in, scale, shift, w_taps, ksize, cin, cout, prenorm):
    T = y_in.shape[0]
    P = ksize // 2
    Kp = w_taps.shape[1]

    def body(yp_ref, yc_ref, yn_ref, sc_ref, sh_ref, w_ref, yo_ref, st_ref,
             xn, xe):
        t = pl.program_id(0)
        b = pl.program_id(1)
        colmask = (lax.broadcasted_iota(jnp.int32, (1, 512, 1), 1) >= 4) & \
                  (lax.broadcasted_iota(jnp.int32, (1, 512, 1), 1) < 504)

        def nr(v):
            f = v.astype(jnp.float32)
            if prenorm:
                f = jnp.maximum(f * sc_ref[...] + sh_ref[...], 0.0)
                f = jnp.where(colmask, f, 0.0)
            return f.astype(jnp.bfloat16)

        xn[0:P] = nr(yp_ref[0, _R - P:_R])
        xn[P:P + _R] = nr(yc_ref[0])
        xn[P + _R:P + _R + P] = nr(yn_ref[0, 0:P])

        @pl.when(b == 0)
        def _():
            xn[0:P] = jnp.zeros((P, 512, cin), jnp.bfloat16)

        @pl.when(b == _NB - 1)
        def _():
            xn[P + _R:P + _R + P] = jnp.zeros((P, 512, cin), jnp.bfloat16)

        x = xn[...]
        taps = []
        for dx in range(ksize):
            s = max(P - dx, 0)
            src0 = max(dx - P, 0)
            L = 512 - abs(dx - P)
            tap = lax.slice_in_dim(x, src0, src0 + L, axis=1)
            parts = []
            if s:
                parts.append(jnp.zeros((_R + 2 * P, s, cin), jnp.bfloat16))
            parts.append(tap)
            if 512 - s - L:
                parts.append(jnp.zeros((_R + 2 * P, 512 - s - L, cin), jnp.bfloat16))
            taps.append(jnp.concatenate(parts, axis=1) if len(parts) > 1 else tap)
        if cin == 64:
            chunks = []
            for i in range(0, ksize, 2):
                if i + 1 < ksize:
                    chunks.append(jnp.concatenate([taps[i], taps[i + 1]], axis=2))
                else:
                    chunks.append(jnp.concatenate(
                        [taps[i], jnp.zeros((_R + 2 * P, 512, 64), jnp.bfloat16)],
                        axis=2))
        else:
            chunks = taps
        for i, ch in enumerate(chunks):
            xe[:, :, i * 128:(i + 1) * 128] = ch

        acc = jnp.zeros((_R * 512, cout), jnp.float32)
        for dy in range(ksize):
            lhs = xe[dy:dy + _R].reshape(_R * 512, Kp)
            acc = acc + jnp.dot(lhs, w_ref[dy], preferred_element_type=jnp.float32)

        a3 = acc.reshape(_R, 512, cout)
        am = jnp.where(colmask, a3, 0.0)
        s1 = jnp.sum(am, axis=(0, 1))
        s2 = jnp.sum(am * a3, axis=(0, 1))
        yo_ref[0] = am.astype(jnp.bfloat16)
        st = jnp.concatenate(
            [s1[None], s2[None], jnp.zeros((6, cout), jnp.float32)], axis=0)
        first = (t == 0) & (b == 0)
        st_ref[...] = jnp.where(first, st, st_ref[...] + st)

    nblk = lambda f: lambda t, b: (t, f(b), 0, 0)
    return pl.pallas_call(
        body,
        grid=(T, _NB),
        in_specs=[
            pl.BlockSpec((1, _R, 512, cin), nblk(lambda b: jnp.maximum(b - 1, 0))),
            pl.BlockSpec((1, _R, 512, cin), nblk(lambda b: b)),
            pl.BlockSpec((1, _R, 512, cin), nblk(lambda b: jnp.minimum(b + 1, _NB - 1))),
            pl.BlockSpec((cin,), lambda t, b: (0,)),
            pl.BlockSpec((cin,), lambda t, b: (0,)),
            pl.BlockSpec((ksize, Kp, cout), lambda t, b: (0, 0, 0)),
        ],
        out_specs=[
            pl.BlockSpec((1, _R, 512, cout), nblk(lambda b: b)),
            pl.BlockSpec((8, cout), lambda t, b: (0, 0)),
        ],
        out_shape=[
            jax.ShapeDtypeStruct((T, _MAP_W, 512, cout), jnp.bfloat16),
            jax.ShapeDtypeStruct((8, cout), jnp.float32),
        ],
        scratch_shapes=[
            pltpu.VMEM((_R + 2 * P, 512, cin), jnp.bfloat16),
            pltpu.VMEM((_R + 2 * P, 512, Kp), jnp.bfloat16),
        ],
        compiler_params=pltpu.CompilerParams(vmem_limit_bytes=60 * 1024 * 1024),
    )(y_in, y_in, y_in, scale, shift, w_taps)


def _l5_body(y_ref, sc_ref, sh_ref, w_ref, b_ref, out_ref):
    colmask = (lax.broadcasted_iota(jnp.int32, (1, 512, 1), 1) >= 4) & \
              (lax.broadcasted_iota(jnp.int32, (1, 512, 1), 1) < 504)
    f = y_ref[0].astype(jnp.float32)
    f = jnp.maximum(f * sc_ref[...] + sh_ref[...], 0.0)
    f = jnp.where(colmask, f, 0.0).astype(jnp.bfloat16)
    acc = jnp.dot(f.reshape(_R * 512, 64), w_ref[...],
                  preferred_element_type=jnp.float32)
    out_ref[0] = acc.reshape(_R, 512, 21) + b_ref[...]


def _bn_ss(stats, g, b, n):
    mu = stats[0] / n
    var = jnp.maximum(stats[1] / n - mu * mu, 0.0)
    sc = g * jax.lax.rsqrt(var + 1e-5)
    return sc, b - mu * sc


def _decoder(mem, w1, g1, b1, w2, g2, b2, w3, g3, b3, w4, g4, b4, w5, b5):
    T = mem.shape[0]
    n = jnp.float32(T * _MAP_W * _MAP_W)
    pad64 = lambda v: jnp.pad(v, (0, 64 - v.shape[0]))
    W1 = _prep_w(w1, 7, 64, 128)
    W2 = _prep_w(w2, 3, 128, 64)
    W3 = _prep_w(w3, 3, 64, 64)
    W4 = _prep_w(w4, 3, 64, 64)
    W5 = jnp.pad(jnp.transpose(w5[:, :, 0, 0], (1, 0)), ((0, 16), (0, 0))).astype(jnp.bfloat16)

    one64 = jnp.ones((64,), jnp.float32)
    zero64 = jnp.zeros((64,), jnp.float32)
    y1, st1 = _conv_layer(mem, one64, zero64, W1, 7, 64, 128, False)
    sc, sh = _bn_ss(st1, g1, b1, n)
    y2, st2 = _conv_layer(y1, sc, sh, W2, 3, 128, 64, True)
    sc, sh = _bn_ss(st2, g2, b2, n)
    y3, st3 = _conv_layer(y2, sc, sh, W3, 3, 64, 64, True)
    sc, sh = _bn_ss(st3, pad64(g3), pad64(b3), n)
    y4, st4 = _conv_layer(y3, sc, sh, W4, 3, 64, 64, True)
    sc, sh = _bn_ss(st4, pad64(g4), pad64(b4), n)
    out = pl.pallas_call(
        _l5_body,
        grid=(T, _NB),
        in_specs=[
            pl.BlockSpec((1, _R, 512, 64), lambda t, b: (t, b, 0, 0)),
            pl.BlockSpec((64,), lambda t, b: (0,)),
            pl.BlockSpec((64,), lambda t, b: (0,)),
            pl.BlockSpec((64, 21), lambda t, b: (0, 0)),
            pl.BlockSpec((21,), lambda t, b: (0,)),
        ],
        out_specs=pl.BlockSpec((1, _R, 512, 21), lambda t, b: (t, b, 0, 0)),
        out_shape=jax.ShapeDtypeStruct((T, _MAP_W, 512, 21), jnp.float32),
    )(y4, sc, sh, W5, b5)
    return jnp.transpose(out[:, :, 4:504, :], (0, 3, 1, 2))


def kernel(features, proj_indices, masks_inliers, W_lin, b_lin, w1, g1, bb1,
           w2, g2, bb2, w3, g3, bb3, w4, g4, bb4, w5, b5):
    T = features.shape[1]
    thr = jnp.max(proj_indices, axis=1, keepdims=True)
    m = proj_indices < thr

    feat = _resize_feat(features[0])
    feat = jnp.transpose(feat, (0, 2, 3, 1))
    feat_flat = feat.reshape(T, _NPIX, 32)

    mflat = masks_inliers.reshape(T, -1).astype(jnp.int32)
    rank = jnp.cumsum(mflat, axis=1) - mflat
    n_inl = jnp.sum(mflat, axis=1)
    j = jnp.arange(_NPIX, dtype=jnp.int32)
    scat_idx = jnp.where((mflat > 0) & (rank < _QCAP), rank, _QCAP)
    pos = jnp.zeros((T, _QCAP + 1), jnp.int32)
    pos = pos.at[jnp.arange(T)[:, None], scat_idx].set(
        jnp.broadcast_to(j[None, :], (T, _NPIX)), mode='drop')

    q = jnp.clip(jnp.minimum(proj_indices, (n_inl - 1)[:, None]), 0, _QCAP - 1)
    q_pad = jnp.pad(q.astype(jnp.int32), ((0, 0), (0, _NW * _CW - _NCELL)))
    rows = _sc_gather(q_pad, pos[:, :_QCAP], feat_flat.reshape(T * _NPIX, 32))

    mem = _project_mask(rows, m.astype(jnp.float32), W_lin, b_lin)
    semmap = _decoder(mem, w1, g1, bb1, w2, g2, bb2, w3, g3, bb3,
                      w4, g4, bb4, w5, b5)
    observed_masks = m.reshape(T, _MAP_W, _MAP_W)
    return (semmap, observed_masks)
